# baseline (speedup 1.0000x reference)
import functools

import jax
import jax.numpy as jnp
from jax import lax
from jax.experimental import pallas as pl
from jax.experimental.pallas import tpu as pltpu
from jax.experimental.pallas import tpu_sc as plsc

_CHUNK = 128
_LANES = 16


def _sc_embed_sum(species_idx, ability_idx, item_idx, move_flat,
                  species_table, ability_table, item_table, action_table):
    B = species_idx.shape[0]
    D = species_table.shape[1]
    info = plsc.get_sparse_core_info()
    nw = info.num_cores * info.num_subcores
    per_w = B // nw
    nchunk = per_w // _CHUNK

    mesh = plsc.VectorSubcoreMesh(core_axis_name="c", subcore_axis_name="s")

    @functools.partial(
        pl.kernel,
        out_type=jax.ShapeDtypeStruct((B, D), jnp.float32),
        mesh=mesh,
        compiler_params=pltpu.CompilerParams(use_tc_tiling_on_sc=False),
        scratch_types=[
            pltpu.VMEM((per_w,), jnp.int32),
            pltpu.VMEM((per_w,), jnp.int32),
            pltpu.VMEM((per_w,), jnp.int32),
            pltpu.VMEM((4 * per_w,), jnp.int32),
            *[pltpu.VMEM((_CHUNK, D), jnp.float32) for _ in range(7)],
            pltpu.SemaphoreType.DMA,
        ],
    )
    def k(sp_hbm, ab_hbm, it_hbm, mv_hbm, spt, abt, itt, act, out_hbm,
          sv, av, iv, mv, b0, b1, b2, b3, b4, b5, b6, sem):
        cid = lax.axis_index("c")
        sid = lax.axis_index("s")
        wid = sid * info.num_cores + cid
        base = wid * per_w
        pltpu.sync_copy(sp_hbm.at[pl.ds(base, per_w)], sv)
        pltpu.sync_copy(ab_hbm.at[pl.ds(base, per_w)], av)
        pltpu.sync_copy(it_hbm.at[pl.ds(base, per_w)], iv)
        for j in range(4):
            pltpu.sync_copy(mv_hbm.at[pl.ds(j * B + base, per_w)],
                            mv.at[pl.ds(j * per_w, per_w)])

        bufs = (b0, b1, b2, b3, b4, b5, b6)
        for c in range(nchunk):
            cb = c * _CHUNK
            cops = [
                pltpu.async_copy(spt.at[sv.at[pl.ds(cb, _CHUNK)]], b0, sem),
                pltpu.async_copy(abt.at[av.at[pl.ds(cb, _CHUNK)]], b1, sem),
                pltpu.async_copy(itt.at[iv.at[pl.ds(cb, _CHUNK)]], b2, sem),
            ]
            for j in range(4):
                cops.append(pltpu.async_copy(
                    act.at[mv.at[pl.ds(j * per_w + cb, _CHUNK)]],
                    bufs[3 + j], sem))
            for cop in cops:
                cop.wait()

            def row_body(i, carry):
                for s4 in range(D // _LANES):
                    sl = pl.ds(s4 * _LANES, _LANES)
                    v = b0[i, sl]
                    for bb in (b1, b2, b3, b4, b5, b6):
                        v = v + bb[i, sl]
                    b0[i, sl] = v
                return carry

            lax.fori_loop(0, _CHUNK, row_body, 0)
            pltpu.sync_copy(b0, out_hbm.at[pl.ds(base + cb, _CHUNK)])

    return k(species_idx, ability_idx, item_idx, move_flat,
             species_table, ability_table, item_table, action_table)


def _mlp_body(emb_ref, w_ref, b_ref, s_ref, o_ref):
    h = jnp.dot(emb_ref[...], w_ref[...], preferred_element_type=jnp.float32)
    h = jnp.maximum(h + b_ref[...], 0.0)
    mask = s_ref[...] != 0
    o_ref[...] = jnp.where(mask, h, 0.0)


def _tc_mlp(emb, W, b, species_idx):
    B, D = emb.shape
    blk = 2048
    return pl.pallas_call(
        _mlp_body,
        grid=(B // blk,),
        in_specs=[
            pl.BlockSpec((blk, D), lambda i: (i, 0)),
            pl.BlockSpec((D, D), lambda i: (0, 0)),
            pl.BlockSpec((1, D), lambda i: (0, 0)),
            pl.BlockSpec((blk, 1), lambda i: (i, 0)),
        ],
        out_specs=pl.BlockSpec((blk, D), lambda i: (i, 0)),
        out_shape=jax.ShapeDtypeStruct((B, D), jnp.float32),
    )(emb, W, b.reshape(1, D), species_idx.reshape(B, 1))


def kernel(species_idx, ability_idx, item_idx, move_idx,
           species_table, ability_table, item_table, action_table, W, b):
    move_flat = move_idx.T.reshape(-1)
    emb = _sc_embed_sum(species_idx, ability_idx, item_idx, move_flat,
                        species_table, ability_table, item_table, action_table)
    return _tc_mlp(emb, W, b, species_idx)
